# baseline (speedup 1.0000x reference)
import functools

import jax
import jax.numpy as jnp
from jax import lax
from jax.experimental import pallas as pl
from jax.experimental.pallas import tpu as pltpu
from jax.experimental.pallas import tpu_sc as plsc

_B = 16384
_GD = 32
_MD = 128
_NC, _NS = 2, 16
_NW = _NC * _NS
_BPW = _B // _NW
_CH = 128
_NCH = _BPW // _CH
_ROWS = _B // _CH
_MHALF = _NCH // 2


def _sc_gather_body(uidx, iidx, ugmf, igmf, umlp, imlp,
                    ug_out, ig_out, um_out, im_out,
                    uidx_v, iidx_v, ug_v, ig_v, um_v, im_v, gsem, msem):
    wid = lax.axis_index("s") * _NC + lax.axis_index("c")
    row0 = wid * _NCH
    pltpu.sync_copy(uidx.at[pl.ds(row0, _NCH)], uidx_v)
    pltpu.sync_copy(iidx.at[pl.ds(row0, _NCH)], iidx_v)
    gmf_cps = []
    for k in range(_NCH):
        gmf_cps.append(pltpu.async_copy(ugmf.at[uidx_v.at[k]], ug_v.at[k], gsem))
        gmf_cps.append(pltpu.async_copy(igmf.at[iidx_v.at[k]], ig_v.at[k], gsem))
    mlp_cps = []
    for k in range(_MHALF):
        mlp_cps.append(pltpu.async_copy(umlp.at[uidx_v.at[k]], um_v.at[k], msem))
        mlp_cps.append(pltpu.async_copy(imlp.at[iidx_v.at[k]], im_v.at[k], msem))
    for cp in gmf_cps:
        cp.wait()
    pltpu.sync_copy(ug_v, ug_out.at[pl.ds(row0, _NCH)])
    pltpu.sync_copy(ig_v, ig_out.at[pl.ds(row0, _NCH)])
    for cp in mlp_cps:
        cp.wait()
    pltpu.sync_copy(um_v, um_out.at[pl.ds(row0, _MHALF)])
    pltpu.sync_copy(im_v, im_out.at[pl.ds(row0, _MHALF)])
    mlp_cps = []
    for k in range(_MHALF):
        mlp_cps.append(pltpu.async_copy(umlp.at[uidx_v.at[_MHALF + k]], um_v.at[k], msem))
        mlp_cps.append(pltpu.async_copy(imlp.at[iidx_v.at[_MHALF + k]], im_v.at[k], msem))
    for cp in mlp_cps:
        cp.wait()
    pltpu.sync_copy(um_v, um_out.at[pl.ds(row0 + _MHALF, _MHALF)])
    pltpu.sync_copy(im_v, im_out.at[pl.ds(row0 + _MHALF, _MHALF)])


_sc_gather = functools.partial(
    pl.kernel,
    out_type=(
        jax.ShapeDtypeStruct((_ROWS, _CH, _GD), jnp.float32),
        jax.ShapeDtypeStruct((_ROWS, _CH, _GD), jnp.float32),
        jax.ShapeDtypeStruct((_ROWS, _CH, _MD), jnp.float32),
        jax.ShapeDtypeStruct((_ROWS, _CH, _MD), jnp.float32),
    ),
    mesh=plsc.VectorSubcoreMesh(core_axis_name="c", subcore_axis_name="s"),
    compiler_params=pltpu.CompilerParams(use_tc_tiling_on_sc=False),
    scratch_types=[
        pltpu.VMEM((_NCH, _CH), jnp.int32),
        pltpu.VMEM((_NCH, _CH), jnp.int32),
        pltpu.VMEM((_NCH, _CH, _GD), jnp.float32),
        pltpu.VMEM((_NCH, _CH, _GD), jnp.float32),
        pltpu.VMEM((_MHALF, _CH, _MD), jnp.float32),
        pltpu.VMEM((_MHALF, _CH, _MD), jnp.float32),
        pltpu.SemaphoreType.DMA,
        pltpu.SemaphoreType.DMA,
    ],
)(_sc_gather_body)


def _tc_mlp_body(ug, ig, um, im, w1u, w1i, b1, w2, b2, w3, b3, wf, bf, out):
    h = jnp.dot(um[...], w1u[...], preferred_element_type=jnp.float32)
    h = h + jnp.dot(im[...], w1i[...], preferred_element_type=jnp.float32)
    h = jnp.maximum(h + b1[...], 0.0)
    h = jnp.maximum(jnp.dot(h, w2[...], preferred_element_type=jnp.float32) + b2[...], 0.0)
    h = jnp.maximum(jnp.dot(h, w3[...], preferred_element_type=jnp.float32) + b3[...], 0.0)
    g = ug[...] * ig[...]
    r = jnp.sum(g * wf[:, :_GD], axis=1) + jnp.sum(h * wf[:, _GD:], axis=1)
    out[...] = r + bf[0, 0]


def _tc_mlp(ug, ig, um, im, w1u, w1i, b1, w2, b2, w3, b3, wf, bf):
    blk = 2048
    grid = (_B // blk,)
    fixed = lambda shape: pl.BlockSpec(shape, lambda i: (0,) * len(shape))
    return pl.pallas_call(
        _tc_mlp_body,
        grid=grid,
        in_specs=[
            pl.BlockSpec((blk, _GD), lambda i: (i, 0)),
            pl.BlockSpec((blk, _GD), lambda i: (i, 0)),
            pl.BlockSpec((blk, _MD), lambda i: (i, 0)),
            pl.BlockSpec((blk, _MD), lambda i: (i, 0)),
            fixed((_MD, _MD)),
            fixed((_MD, _MD)),
            fixed((1, _MD)),
            fixed((_MD, 64)),
            fixed((1, 64)),
            fixed((64, _GD)),
            fixed((1, _GD)),
            fixed((1, 2 * _GD)),
            fixed((1, 1)),
        ],
        out_specs=pl.BlockSpec((blk,), lambda i: (i,)),
        out_shape=jax.ShapeDtypeStruct((_B,), jnp.float32),
    )(ug, ig, um, im, w1u, w1i, b1, w2, b2, w3, b3, wf, bf)


def kernel(user_indices, item_indices, user_gmf_table, item_gmf_table,
           user_mlp_table, item_mlp_table, W1, b1, W2, b2, W3, b3, Wf, bf):
    uidx = user_indices.reshape(_ROWS, _CH)
    iidx = item_indices.reshape(_ROWS, _CH)
    ug, ig, um, im = _sc_gather(uidx, iidx, user_gmf_table, item_gmf_table,
                                user_mlp_table, item_mlp_table)
    ug = ug.reshape(_B, _GD)
    ig = ig.reshape(_B, _GD)
    um = um.reshape(_B, _MD)
    im = im.reshape(_B, _MD)
    w1u = W1[:, :_MD].T
    w1i = W1[:, _MD:].T
    return _tc_mlp(ug, ig, um, im, w1u, w1i, b1.reshape(1, _MD),
                   W2.T, b2.reshape(1, 64), W3.T, b3.reshape(1, _GD),
                   Wf, bf.reshape(1, 1))


# speedup vs baseline: 1.0045x; 1.0045x over previous
import functools

import jax
import jax.numpy as jnp
from jax import lax
from jax.experimental import pallas as pl
from jax.experimental.pallas import tpu as pltpu
from jax.experimental.pallas import tpu_sc as plsc

_B = 16384
_GD = 32
_MD = 128
_NC, _NS = 2, 16
_NW = _NC * _NS
_BPW = _B // _NW
_CH = 128
_NCH = _BPW // _CH
_ROWS = _B // _CH
_MHALF = _NCH // 2

_MESH = plsc.VectorSubcoreMesh(core_axis_name="c", subcore_axis_name="s")


def _worker_id():
    return lax.axis_index("s") * _NC + lax.axis_index("c")


def _sc_gmf_body(uidx, iidx, ugmf, igmf, ug_out, ig_out,
                 uidx_v, iidx_v, ug_v, ig_v, sem):
    wid = _worker_id()
    row0 = wid * _NCH
    pltpu.sync_copy(uidx.at[pl.ds(row0, _NCH)], uidx_v)
    pltpu.sync_copy(iidx.at[pl.ds(row0, _NCH)], iidx_v)
    cps = []
    for k in range(_NCH):
        cps.append(pltpu.async_copy(ugmf.at[uidx_v.at[k]], ug_v.at[k], sem))
        cps.append(pltpu.async_copy(igmf.at[iidx_v.at[k]], ig_v.at[k], sem))
    for cp in cps:
        cp.wait()
    pltpu.sync_copy(ug_v, ug_out.at[pl.ds(row0, _NCH)])
    pltpu.sync_copy(ig_v, ig_out.at[pl.ds(row0, _NCH)])


_sc_gmf = functools.partial(
    pl.kernel,
    out_type=(
        jax.ShapeDtypeStruct((_ROWS, _CH, _GD), jnp.float32),
        jax.ShapeDtypeStruct((_ROWS, _CH, _GD), jnp.float32),
    ),
    mesh=_MESH,
    compiler_params=pltpu.CompilerParams(use_tc_tiling_on_sc=False),
    scratch_types=[
        pltpu.VMEM((_NCH, _CH), jnp.int32),
        pltpu.VMEM((_NCH, _CH), jnp.int32),
        pltpu.VMEM((_NCH, _CH, _GD), jnp.float32),
        pltpu.VMEM((_NCH, _CH, _GD), jnp.float32),
        pltpu.SemaphoreType.DMA,
    ],
)(_sc_gmf_body)


def _sc_mlp_body(uidx, iidx, umlp, imlp, um_out, im_out,
                 uidx_v, iidx_v, um_v, im_v, sem):
    wid = _worker_id()
    base = wid * _BPW
    pltpu.sync_copy(uidx.at[pl.ds(base, _BPW)], uidx_v)
    pltpu.sync_copy(iidx.at[pl.ds(base, _BPW)], iidx_v)
    half = _MHALF * _CH
    for h in range(2):
        cps = []
        for k in range(_MHALF):
            off = h * half + k * _CH
            cps.append(pltpu.async_copy(
                umlp.at[uidx_v.at[pl.ds(off, _CH)]],
                um_v.at[pl.ds(k * _CH, _CH)], sem))
            cps.append(pltpu.async_copy(
                imlp.at[iidx_v.at[pl.ds(off, _CH)]],
                im_v.at[pl.ds(k * _CH, _CH)], sem))
        for cp in cps:
            cp.wait()
        pltpu.sync_copy(um_v, um_out.at[pl.ds(base + h * half, half)])
        pltpu.sync_copy(im_v, im_out.at[pl.ds(base + h * half, half)])


_sc_mlp = functools.partial(
    pl.kernel,
    out_type=(
        jax.ShapeDtypeStruct((_B, _MD), jnp.float32),
        jax.ShapeDtypeStruct((_B, _MD), jnp.float32),
    ),
    mesh=_MESH,
    scratch_types=[
        pltpu.VMEM((_BPW,), jnp.int32),
        pltpu.VMEM((_BPW,), jnp.int32),
        pltpu.VMEM((_MHALF * _CH, _MD), jnp.float32),
        pltpu.VMEM((_MHALF * _CH, _MD), jnp.float32),
        pltpu.SemaphoreType.DMA,
    ],
)(_sc_mlp_body)


def _tc_mlp_body(ug, ig, um, im, w1u, w1i, b1, w2, b2, w3, b3, wf, bf, out):
    h = jnp.dot(um[...], w1u[...], preferred_element_type=jnp.float32)
    h = h + jnp.dot(im[...], w1i[...], preferred_element_type=jnp.float32)
    h = jnp.maximum(h + b1[...], 0.0)
    h = jnp.maximum(jnp.dot(h, w2[...], preferred_element_type=jnp.float32) + b2[...], 0.0)
    h = jnp.maximum(jnp.dot(h, w3[...], preferred_element_type=jnp.float32) + b3[...], 0.0)
    g = ug[...] * ig[...]
    r = jnp.sum(g * wf[:, :_GD], axis=1) + jnp.sum(h * wf[:, _GD:], axis=1)
    out[...] = r + bf[0, 0]


def _tc_mlp(ug, ig, um, im, w1u, w1i, b1, w2, b2, w3, b3, wf, bf):
    blk = 2048
    grid = (_B // blk,)
    fixed = lambda shape: pl.BlockSpec(shape, lambda i: (0,) * len(shape))
    return pl.pallas_call(
        _tc_mlp_body,
        grid=grid,
        in_specs=[
            pl.BlockSpec((blk, _GD), lambda i: (i, 0)),
            pl.BlockSpec((blk, _GD), lambda i: (i, 0)),
            pl.BlockSpec((blk, _MD), lambda i: (i, 0)),
            pl.BlockSpec((blk, _MD), lambda i: (i, 0)),
            fixed((_MD, _MD)),
            fixed((_MD, _MD)),
            fixed((1, _MD)),
            fixed((_MD, 64)),
            fixed((1, 64)),
            fixed((64, _GD)),
            fixed((1, _GD)),
            fixed((1, 2 * _GD)),
            fixed((1, 1)),
        ],
        out_specs=pl.BlockSpec((blk,), lambda i: (i,)),
        out_shape=jax.ShapeDtypeStruct((_B,), jnp.float32),
    )(ug, ig, um, im, w1u, w1i, b1, w2, b2, w3, b3, wf, bf)


def kernel(user_indices, item_indices, user_gmf_table, item_gmf_table,
           user_mlp_table, item_mlp_table, W1, b1, W2, b2, W3, b3, Wf, bf):
    uidx2 = user_indices.reshape(_ROWS, _CH)
    iidx2 = item_indices.reshape(_ROWS, _CH)
    ug, ig = _sc_gmf(uidx2, iidx2, user_gmf_table, item_gmf_table)
    um, im = _sc_mlp(user_indices, item_indices, user_mlp_table, item_mlp_table)
    ug = ug.reshape(_B, _GD)
    ig = ig.reshape(_B, _GD)
    w1u = W1[:, :_MD].T
    w1i = W1[:, _MD:].T
    return _tc_mlp(ug, ig, um, im, w1u, w1i, b1.reshape(1, _MD),
                   W2.T, b2.reshape(1, 64), W3.T, b3.reshape(1, _GD),
                   Wf, bf.reshape(1, 1))


# speedup vs baseline: 4.6887x; 4.6678x over previous
import functools

import jax
import jax.numpy as jnp
from jax import lax
from jax.experimental import pallas as pl
from jax.experimental.pallas import tpu as pltpu
from jax.experimental.pallas import tpu_sc as plsc

_B = 16384
_GD = 32
_MD = 128
_NC, _NS = 2, 16
_NW = _NC * _NS
_BPW = _B // _NW
_CH = 128
_NCH = _BPW // _CH
_ROWS = _B // _CH
_MHALF = _NCH // 2

_MESH = plsc.VectorSubcoreMesh(core_axis_name="c", subcore_axis_name="s")


def _worker_id():
    return lax.axis_index("s") * _NC + lax.axis_index("c")


def _sc_gmf_body(uidx, iidx, ugmf, igmf, ug_out, ig_out,
                 uidx_v, iidx_v, ug_v, ig_v, sem):
    wid = _worker_id()
    row0 = wid * _NCH
    pltpu.sync_copy(uidx.at[pl.ds(row0, _NCH)], uidx_v)
    pltpu.sync_copy(iidx.at[pl.ds(row0, _NCH)], iidx_v)
    cps = []
    for k in range(_NCH):
        cps.append(pltpu.async_copy(ugmf.at[uidx_v.at[k]], ug_v.at[k], sem))
        cps.append(pltpu.async_copy(igmf.at[iidx_v.at[k]], ig_v.at[k], sem))
    for cp in cps:
        cp.wait()
    pltpu.sync_copy(ug_v, ug_out.at[pl.ds(row0, _NCH)])
    pltpu.sync_copy(ig_v, ig_out.at[pl.ds(row0, _NCH)])


_sc_gmf = functools.partial(
    pl.kernel,
    out_type=(
        jax.ShapeDtypeStruct((_ROWS, _CH, _GD), jnp.float32),
        jax.ShapeDtypeStruct((_ROWS, _CH, _GD), jnp.float32),
    ),
    mesh=_MESH,
    compiler_params=pltpu.CompilerParams(use_tc_tiling_on_sc=False),
    scratch_types=[
        pltpu.VMEM((_NCH, _CH), jnp.int32),
        pltpu.VMEM((_NCH, _CH), jnp.int32),
        pltpu.VMEM((_NCH, _CH, _GD), jnp.float32),
        pltpu.VMEM((_NCH, _CH, _GD), jnp.float32),
        pltpu.SemaphoreType.DMA,
    ],
)(_sc_gmf_body)


def _sc_mlp_body(uidx, iidx, umlp, imlp, um_out, im_out,
                 uidx_v, iidx_v, um_v, im_v, sem):
    wid = _worker_id()
    base = wid * _BPW
    pltpu.sync_copy(uidx.at[pl.ds(base, _BPW)], uidx_v)
    pltpu.sync_copy(iidx.at[pl.ds(base, _BPW)], iidx_v)
    half = _MHALF * _CH
    for h in range(2):
        cps = []
        for k in range(_MHALF):
            off = h * half + k * _CH
            cps.append(pltpu.async_copy(
                umlp.at[uidx_v.at[pl.ds(off, _CH)]],
                um_v.at[pl.ds(k * _CH, _CH)], sem))
            cps.append(pltpu.async_copy(
                imlp.at[iidx_v.at[pl.ds(off, _CH)]],
                im_v.at[pl.ds(k * _CH, _CH)], sem))
        for cp in cps:
            cp.wait()
        pltpu.sync_copy(um_v, um_out.at[pl.ds(base + h * half, half)])
        pltpu.sync_copy(im_v, im_out.at[pl.ds(base + h * half, half)])


_sc_mlp = functools.partial(
    pl.kernel,
    out_type=(
        jax.ShapeDtypeStruct((_B, _MD), jnp.float32),
        jax.ShapeDtypeStruct((_B, _MD), jnp.float32),
    ),
    mesh=_MESH,
    scratch_types=[
        pltpu.VMEM((_BPW,), jnp.int32),
        pltpu.VMEM((_BPW,), jnp.int32),
        pltpu.VMEM((_MHALF * _CH, _MD), jnp.float32),
        pltpu.VMEM((_MHALF * _CH, _MD), jnp.float32),
        pltpu.SemaphoreType.DMA,
    ],
)(_sc_mlp_body)


def _tc_mlp_body(ug, ig, um, im, w1u, w1i, b1, w2, b2, w3, b3, wf, bf, out):
    h = jnp.dot(um[...], w1u[...], preferred_element_type=jnp.float32)
    h = h + jnp.dot(im[...], w1i[...], preferred_element_type=jnp.float32)
    h = jnp.maximum(h + b1[...], 0.0)
    h = jnp.maximum(jnp.dot(h, w2[...], preferred_element_type=jnp.float32) + b2[...], 0.0)
    h = jnp.maximum(jnp.dot(h, w3[...], preferred_element_type=jnp.float32) + b3[...], 0.0)
    g = ug[...] * ig[...]
    r = jnp.sum(g * wf[:, :_GD], axis=1) + jnp.sum(h * wf[:, _GD:], axis=1)
    out[...] = r + bf[0, 0]


def _tc_mlp(ug, ig, um, im, w1u, w1i, b1, w2, b2, w3, b3, wf, bf):
    blk = 2048
    grid = (_B // blk,)
    fixed = lambda shape: pl.BlockSpec(shape, lambda i: (0,) * len(shape))
    return pl.pallas_call(
        _tc_mlp_body,
        grid=grid,
        in_specs=[
            pl.BlockSpec((blk, _GD), lambda i: (i, 0)),
            pl.BlockSpec((blk, _GD), lambda i: (i, 0)),
            pl.BlockSpec((blk, _MD), lambda i: (i, 0)),
            pl.BlockSpec((blk, _MD), lambda i: (i, 0)),
            fixed((_MD, _MD)),
            fixed((_MD, _MD)),
            fixed((1, _MD)),
            fixed((_MD, 64)),
            fixed((1, 64)),
            fixed((64, _GD)),
            fixed((1, _GD)),
            fixed((1, 2 * _GD)),
            fixed((1, 1)),
        ],
        out_specs=pl.BlockSpec((blk,), lambda i: (i,)),
        out_shape=jax.ShapeDtypeStruct((_B,), jnp.float32),
    )(ug, ig, um, im, w1u, w1i, b1, w2, b2, w3, b3, wf, bf)


def kernel(user_indices, item_indices, user_gmf_table, item_gmf_table,
           user_mlp_table, item_mlp_table, W1, b1, W2, b2, W3, b3, Wf, bf):
    ug = jnp.take(user_gmf_table, user_indices, axis=0)
    ig = jnp.take(item_gmf_table, item_indices, axis=0)
    um, im = _sc_mlp(user_indices, item_indices, user_mlp_table, item_mlp_table)
    w1u = W1[:, :_MD].T
    w1i = W1[:, _MD:].T
    return _tc_mlp(ug, ig, um, im, w1u, w1i, b1.reshape(1, _MD),
                   W2.T, b2.reshape(1, 64), W3.T, b3.reshape(1, _GD),
                   Wf, bf.reshape(1, 1))
